# baseline (speedup 1.0000x reference)
import functools

import jax
import jax.numpy as jnp
from jax import lax
from jax.experimental import pallas as pl
from jax.experimental.pallas import tpu as pltpu
from jax.experimental.pallas import tpu_sc as plsc

NC = 2
NS = 16
CH = 128
G = 8


@functools.partial(jax.jit, static_argnums=(4, 5, 6))
def _run(feature, src2, dst2, zeros, N, D, n_chunks):
    nup = -(-(N + 1) // (8 * NS)) * (8 * NS)
    zrows = nup // NS
    n_groups = n_chunks // G

    mesh = plsc.VectorSubcoreMesh(core_axis_name="c", subcore_axis_name="s")

    @functools.partial(
        pl.kernel,
        out_type=jax.ShapeDtypeStruct((NC, nup, D), jnp.float32),
        mesh=mesh,
        scratch_types=[
            pltpu.VMEM_SHARED((nup, D), jnp.float32),
            pltpu.VMEM((CH // 2,), jnp.int32),
            pltpu.VMEM((CH // 2,), jnp.int32),
            pltpu.VMEM((CH,), jnp.int32),
            pltpu.VMEM((CH, D), jnp.float32),
            pltpu.SemaphoreType.DMA,
            pltpu.SemaphoreType.DMA,
        ],
    )
    def k(feat_hbm, src_hbm, dst_hbm, zeros_hbm, part_hbm, acc, src_v1,
          src_v2, dst_v, rows_v, sem1, sem2):
        c = lax.axis_index("c")
        s = lax.axis_index("s")
        wid = s * NC + c
        base = wid * n_chunks * CH
        pltpu.sync_copy(zeros_hbm, acc.at[pl.ds(s * zrows, zrows)])
        plsc.subcore_barrier()
        h = CH // 2

        def step(j, carry):
            off = base + j * CH
            pltpu.sync_copy(src_hbm.at[pl.ds(off, h)], src_v1)
            pltpu.sync_copy(src_hbm.at[pl.ds(off + h, h)], src_v2)
            pltpu.sync_copy(dst_hbm.at[pl.ds(off, CH)], dst_v)
            d1 = pltpu.async_copy(feat_hbm.at[src_v1],
                                  rows_v.at[pl.ds(0, h)], sem1)
            d2 = pltpu.async_copy(feat_hbm.at[src_v2],
                                  rows_v.at[pl.ds(h, h)], sem2)
            d1.wait()
            d2.wait()
            pltpu.sync_copy(rows_v, acc.at[dst_v], add=True)
            return carry

        lax.fori_loop(0, n_chunks, step, 0)
        plsc.subcore_barrier()
        pltpu.sync_copy(acc.at[pl.ds(s * zrows, zrows)],
                        part_hbm.at[c].at[pl.ds(s * zrows, zrows)])

    part = k(feature, src2, dst2, zeros)

    rb = 1000

    def add_body(p_ref, o_ref):
        o_ref[...] = p_ref[0] + p_ref[1]

    return pl.pallas_call(
        add_body,
        grid=(N // rb,),
        in_specs=[pl.BlockSpec((NC, rb, D), lambda i: (0, i, 0))],
        out_specs=pl.BlockSpec((rb, D), lambda i: (i, 0)),
        out_shape=jax.ShapeDtypeStruct((N, D), jnp.float32),
    )(part)


def kernel(feature, edge_index):
    N, D = feature.shape
    E = edge_index.shape[1]
    nw = NC * NS
    n_chunks = -(-(-(-E // nw)) // (G * CH)) * G
    EP = n_chunks * CH * nw
    pad = EP - E
    src = jnp.concatenate(
        [edge_index[0].astype(jnp.int32), jnp.zeros((pad,), jnp.int32)])
    dst = jnp.concatenate(
        [edge_index[1].astype(jnp.int32), jnp.full((pad,), N, jnp.int32)])
    nup = -(-(N + 1) // (8 * NS)) * (8 * NS)
    zeros = jnp.zeros((nup // NS, D), jnp.float32)
    return _run(feature, src, dst, zeros, N, D, n_chunks)


# speedup vs baseline: 1.1526x; 1.1526x over previous
import functools

import jax
import jax.numpy as jnp
from jax import lax
from jax.experimental import pallas as pl
from jax.experimental.pallas import tpu as pltpu
from jax.experimental.pallas import tpu_sc as plsc

NC = 2
NS = 16
CH = 128


@functools.partial(jax.jit, static_argnums=(4, 5, 6))
def _run(feature, src, dst, zeros, N, D, n_chunks):
    nup = -(-(N + 1) // (8 * NS)) * (8 * NS)
    zrows = nup // NS

    mesh = plsc.VectorSubcoreMesh(core_axis_name="c", subcore_axis_name="s")

    @functools.partial(
        pl.kernel,
        out_type=jax.ShapeDtypeStruct((NC, nup, D), jnp.float32),
        mesh=mesh,
        scratch_types=[
            pltpu.VMEM_SHARED((nup, D), jnp.float32),
            pltpu.VMEM((CH,), jnp.int32),
            pltpu.VMEM((CH,), jnp.int32),
            pltpu.VMEM((CH,), jnp.int32),
            pltpu.VMEM((CH,), jnp.int32),
            pltpu.VMEM((CH, D), jnp.float32),
            pltpu.SemaphoreType.DMA,
        ],
    )
    def k(feat_hbm, src_hbm, dst_hbm, zeros_hbm, part_hbm, acc,
          src_a, dst_a, src_b, dst_b, rows_v, sem):
        c = lax.axis_index("c")
        s = lax.axis_index("s")
        wid = s * NC + c
        base = wid * n_chunks * CH
        pltpu.sync_copy(zeros_hbm, acc.at[pl.ds(s * zrows, zrows)])
        pltpu.sync_copy(src_hbm.at[pl.ds(base, CH)], src_a)
        pltpu.sync_copy(dst_hbm.at[pl.ds(base, CH)], dst_a)
        plsc.subcore_barrier()

        def half(off_next, sv, dv, sv_next, dv_next):
            d = pltpu.async_copy(feat_hbm.at[sv], rows_v, sem)
            pltpu.sync_copy(src_hbm.at[pl.ds(off_next, CH)], sv_next)
            pltpu.sync_copy(dst_hbm.at[pl.ds(off_next, CH)], dv_next)
            d.wait()
            pltpu.sync_copy(rows_v, acc.at[dv], add=True)

        def step(i, carry):
            a = base + 2 * i * CH
            half(a + CH, src_a, dst_a, src_b, dst_b)
            half(a + 2 * CH, src_b, dst_b, src_a, dst_a)
            return carry

        lax.fori_loop(0, n_chunks // 2, step, 0)
        plsc.subcore_barrier()
        pltpu.sync_copy(acc.at[pl.ds(s * zrows, zrows)],
                        part_hbm.at[c].at[pl.ds(s * zrows, zrows)])

    part = k(feature, src, dst, zeros)

    rb = 1000

    def add_body(p_ref, o_ref):
        o_ref[...] = p_ref[0] + p_ref[1]

    return pl.pallas_call(
        add_body,
        grid=(N // rb,),
        in_specs=[pl.BlockSpec((NC, rb, D), lambda i: (0, i, 0))],
        out_specs=pl.BlockSpec((rb, D), lambda i: (i, 0)),
        out_shape=jax.ShapeDtypeStruct((N, D), jnp.float32),
    )(part)


def kernel(feature, edge_index):
    N, D = feature.shape
    E = edge_index.shape[1]
    nw = NC * NS
    n_chunks = -(-(-(-E // nw)) // (2 * CH)) * 2
    EP = (n_chunks * nw + 2) * CH
    pad = EP - E
    src = jnp.concatenate(
        [edge_index[0].astype(jnp.int32), jnp.zeros((pad,), jnp.int32)])
    dst = jnp.concatenate(
        [edge_index[1].astype(jnp.int32), jnp.full((pad,), N, jnp.int32)])
    nup = -(-(N + 1) // (8 * NS)) * (8 * NS)
    zeros = jnp.zeros((nup // NS, D), jnp.float32)
    return _run(feature, src, dst, zeros, N, D, n_chunks)


# speedup vs baseline: 1.5124x; 1.3121x over previous
import functools

import jax
import jax.numpy as jnp
from jax import lax
from jax.experimental import pallas as pl
from jax.experimental.pallas import tpu as pltpu
from jax.experimental.pallas import tpu_sc as plsc

NC = 2
NS = 16
CH = 128


@functools.partial(jax.jit, static_argnums=(4, 5, 6))
def _run(feature, src, dst, zeros, N, D, EP):
    per_tile = EP // (NC * NS)
    n_chunks = per_tile // CH
    nup = -(-(N + 1) // (8 * NS)) * (8 * NS)
    zrows = nup // NS

    mesh = plsc.VectorSubcoreMesh(core_axis_name="c", subcore_axis_name="s")

    @functools.partial(
        pl.kernel,
        out_type=jax.ShapeDtypeStruct((NC, nup, D), jnp.float32),
        mesh=mesh,
        scratch_types=[
            pltpu.VMEM_SHARED((nup, D), jnp.float32),
            pltpu.VMEM((CH,), jnp.int32),
            pltpu.VMEM((CH,), jnp.int32),
            pltpu.VMEM((CH, D), jnp.float32),
            pltpu.SemaphoreType.DMA,
        ],
    )
    def k(feat_hbm, src_hbm, dst_hbm, zeros_hbm, part_hbm, acc, src_v, dst_v,
          rows_v, sem):
        c = lax.axis_index("c")
        s = lax.axis_index("s")
        pltpu.sync_copy(zeros_hbm, acc.at[pl.ds(s * zrows, zrows)])
        plsc.subcore_barrier()

        base = (s * NC + c) * per_tile

        def step(i, carry):
            off = base + i * CH
            pltpu.sync_copy(src_hbm.at[pl.ds(off, CH)], src_v)
            pltpu.sync_copy(dst_hbm.at[pl.ds(off, CH)], dst_v)
            pltpu.async_copy(feat_hbm.at[src_v], rows_v, sem).wait()
            pltpu.sync_copy(rows_v, acc.at[dst_v], add=True)
            return carry

        lax.fori_loop(0, n_chunks, step, 0)
        plsc.subcore_barrier()
        pltpu.sync_copy(acc.at[pl.ds(s * zrows, zrows)],
                        part_hbm.at[c].at[pl.ds(s * zrows, zrows)])

    part = k(feature, src, dst, zeros)

    rb = 1000

    def add_body(p_ref, o_ref):
        o_ref[...] = p_ref[0] + p_ref[1]

    return pl.pallas_call(
        add_body,
        grid=(N // rb,),
        in_specs=[pl.BlockSpec((NC, rb, D), lambda i: (0, i, 0))],
        out_specs=pl.BlockSpec((rb, D), lambda i: (i, 0)),
        out_shape=jax.ShapeDtypeStruct((N, D), jnp.float32),
    )(part)


def kernel(feature, edge_index):
    N, D = feature.shape
    E = edge_index.shape[1]
    nw = NC * NS
    per_tile = -(-(-(-E // nw)) // CH) * CH
    EP = per_tile * nw
    pad = EP - E
    src = jnp.concatenate(
        [edge_index[0].astype(jnp.int32), jnp.zeros((pad,), jnp.int32)])
    dst = jnp.concatenate(
        [edge_index[1].astype(jnp.int32), jnp.full((pad,), N, jnp.int32)])
    nup = -(-(N + 1) // (8 * NS)) * (8 * NS)
    zeros = jnp.zeros((nup // NS, D), jnp.float32)
    return _run(feature, src, dst, zeros, N, D, EP)
